# Initial kernel scaffold; baseline (speedup 1.0000x reference)
import jax
import jax.numpy as jnp
from jax.experimental import pallas as pl


def kernel(x, Wg, W1, b1, W2, b2):
    raise NotImplementedError("write your pallas kernel here")


# speedup vs baseline: 1.0147x; 1.0147x over previous
import functools

import jax
import jax.numpy as jnp
from jax.experimental import pallas as pl

B, S, D = 1, 2048, 768
E, K = 16, 2
H = 3072
HC = 768


def _gate_kernel(x_ref, wg_ref, comb_ref, aux_ref):
    x = x_ref[...].astype(jnp.bfloat16)
    wg = wg_ref[...].astype(jnp.bfloat16)
    logits = jax.lax.dot_general(
        x, wg, (((1,), (1,)), ((), ())),
        preferred_element_type=jnp.float32,
    )
    m = jnp.max(logits, axis=-1, keepdims=True)
    ex = jnp.exp(logits - m)
    probs = ex / jnp.sum(ex, axis=-1, keepdims=True)

    lane = jax.lax.broadcasted_iota(jnp.int32, probs.shape, 1)
    m1 = jnp.max(probs, axis=-1, keepdims=True)
    i1 = jnp.min(jnp.where(probs == m1, lane, E), axis=-1, keepdims=True)
    probs2 = jnp.where(lane == i1, -1.0, probs)
    m2 = jnp.max(probs2, axis=-1, keepdims=True)
    i2 = jnp.min(jnp.where(probs2 == m2, lane, E), axis=-1, keepdims=True)

    denom = m1 + m2
    p1 = m1 / denom
    p2 = m2 / denom
    comb = jnp.where(lane == i1, p1, 0.0) + jnp.where(lane == i2, p2, 0.0)
    comb_ref[...] = comb

    t = jnp.float32(probs.shape[0])
    mean_prob = jnp.sum(probs, axis=0, keepdims=True) / t
    ind = (probs > m2).astype(jnp.float32)
    mean_ind = jnp.sum(ind, axis=0, keepdims=True) / t
    aux_ref[...] = jnp.sum(mean_prob * mean_ind, keepdims=True).reshape(1, 1) * E


def _expert_kernel(x_ref, w1_ref, b1_ref, w2_ref, b2_ref, comb_ref, out_ref):
    e = pl.program_id(0)
    x = x_ref[...]
    lane = jax.lax.broadcasted_iota(jnp.int32, comb_ref.shape, 1)
    c = jnp.sum(jnp.where(lane == e, comb_ref[...], 0.0), axis=-1, keepdims=True)

    acc = jnp.zeros((x.shape[0], D), jnp.float32)
    for hc in range(H // HC):
        w1c = w1_ref[0, hc * HC:(hc + 1) * HC, :]
        h = jax.lax.dot_general(
            x, w1c, (((1,), (1,)), ((), ())),
            preferred_element_type=jnp.float32)
        h = h + b1_ref[0, 0, hc * HC:(hc + 1) * HC][None, :]
        h = jnp.maximum(h, 0.0).astype(jnp.bfloat16)
        w2c = w2_ref[0, :, hc * HC:(hc + 1) * HC]
        acc = acc + jax.lax.dot_general(
            h, w2c, (((1,), (1,)), ((), ())),
            preferred_element_type=jnp.float32)
    acc = acc + b2_ref[0, 0, :][None, :]

    @pl.when(e == 0)
    def _():
        out_ref[...] = jnp.zeros_like(out_ref)

    out_ref[...] += c * acc


def kernel(x, Wg, W1, b1, W2, b2):
    x_flat = x.reshape(-1, D)
    T = x_flat.shape[0]

    comb, aux = pl.pallas_call(
        _gate_kernel,
        out_shape=(
            jax.ShapeDtypeStruct((T, E), jnp.float32),
            jax.ShapeDtypeStruct((1, 1), jnp.float32),
        ),
    )(x_flat, Wg)

    x_bf = x_flat.astype(jnp.bfloat16)
    w1_bf = W1.astype(jnp.bfloat16)
    w2_bf = W2.astype(jnp.bfloat16)

    out = pl.pallas_call(
        _expert_kernel,
        grid=(E,),
        in_specs=[
            pl.BlockSpec((T, D), lambda e: (0, 0)),
            pl.BlockSpec((1, H, D), lambda e: (e, 0, 0)),
            pl.BlockSpec((1, 1, H), lambda e: (e, 0, 0)),
            pl.BlockSpec((1, D, H), lambda e: (e, 0, 0)),
            pl.BlockSpec((1, 1, D), lambda e: (e, 0, 0)),
            pl.BlockSpec((T, E), lambda e: (0, 0)),
        ],
        out_specs=pl.BlockSpec((T, D), lambda e: (0, 0)),
        out_shape=jax.ShapeDtypeStruct((T, D), jnp.float32),
    )(x_bf, w1_bf, b1.reshape(E, 1, H), w2_bf, b2.reshape(E, 1, D), comb)

    return out.reshape(B, S, D), aux[0, 0]
